# baseline (speedup 1.0000x reference)
import functools

import jax
import jax.numpy as jnp
from jax import lax
from jax.experimental import pallas as pl
from jax.experimental.pallas import tpu as pltpu
from jax.experimental.pallas import tpu_sc as plsc

N_NODES = 10000
FEAT = 128
NW = 32
CHUNK = 128
CHUNKS = 84
BLK = 4
NBLOCKS = CHUNKS // BLK
PER_TILE = CHUNKS * CHUNK
TOTAL_SLOTS = NW * PER_TILE
NB = 10240
ROWS_PER_TILE = NB // 16
MM_BLK = 1000
GRID = N_NODES // MM_BLK

def _deg_body(cols_hbm, degp_hbm, col_v, hist_v):
    c = lax.axis_index("c")
    s = lax.axis_index("s")
    w = s * 2 + c
    pltpu.sync_copy(cols_hbm.at[w], col_v)

    def zero(i, carry):
        hist_v[pl.ds(i * 16, 16)] = jnp.zeros((16,), jnp.float32)
        return carry

    lax.fori_loop(0, NB // 16, zero, 0)

    ones = jnp.ones((16,), jnp.float32)

    def body(i, carry):
        j = i // (CHUNK // 16)
        l = i % (CHUNK // 16)
        idx = col_v[j, pl.ds(l * 16, 16)]
        plsc.addupdate_scatter(hist_v, [idx], ones)
        return carry

    lax.fori_loop(0, CHUNKS * (CHUNK // 16), body, 0)
    pltpu.sync_copy(hist_v, degp_hbm.at[w])


def _edge_body(rc_hbm, h_hbm, part_hbm, rc_v, buf, acc, gs):
    c = lax.axis_index("c")
    s = lax.axis_index("s")
    w = s * 2 + c
    pltpu.sync_copy(rc_hbm.at[w], rc_v)

    def zero(i, carry):
        buf[i // 8, pl.ds((i % 8) * 16, 16)] = jnp.zeros((16,), jnp.float32)
        return carry

    lax.fori_loop(0, CHUNK * 8, zero, 0)
    for t in range(ROWS_PER_TILE // CHUNK):
        pltpu.sync_copy(buf, acc.at[pl.ds(s * ROWS_PER_TILE + t * CHUNK, CHUNK)])
    plsc.subcore_barrier()

    def body(j, carry):
        pltpu.async_copy(h_hbm.at[rc_v.at[2 * j]], buf, gs).wait()
        pltpu.sync_copy(buf, acc.at[rc_v.at[2 * j + 1]], add=True)
        return carry

    lax.fori_loop(0, CHUNKS, body, 0)
    plsc.subcore_barrier()
    pltpu.sync_copy(
        acc.at[pl.ds(s * ROWS_PER_TILE, ROWS_PER_TILE)],
        part_hbm.at[c].at[pl.ds(s * ROWS_PER_TILE, ROWS_PER_TILE)],
    )


@functools.lru_cache(maxsize=None)
def _sc_calls():
    mesh = plsc.VectorSubcoreMesh(core_axis_name="c", subcore_axis_name="s")
    params = pltpu.CompilerParams(needs_layout_passes=False)
    deg_call = functools.partial(
        pl.kernel,
        out_type=jax.ShapeDtypeStruct((NW, NB), jnp.float32),
        mesh=mesh,
        compiler_params=params,
        scratch_types=[
            pltpu.VMEM((CHUNKS, CHUNK), jnp.int32),
            pltpu.VMEM((NB,), jnp.float32),
        ],
    )(_deg_body)
    edge_call = functools.partial(
        pl.kernel,
        out_type=jax.ShapeDtypeStruct((2, NB, FEAT), jnp.float32),
        mesh=mesh,
        compiler_params=params,
        scratch_types=[
            pltpu.VMEM((2 * CHUNKS, CHUNK), jnp.int32),
            pltpu.VMEM((CHUNK, FEAT), jnp.float32),
            pltpu.VMEM_SHARED((NB, FEAT), jnp.float32),
            pltpu.SemaphoreType.DMA,
        ],
    )(_edge_body)
    return deg_call, edge_call


def _mm_body(x_ref, w_ref, degp_ref, h_ref):
    deg = jnp.sum(degp_ref[...], axis=1)
    dinv = lax.rsqrt(deg)
    h = jnp.dot(x_ref[...], w_ref[...], preferred_element_type=jnp.float32)
    h_ref[...] = h * dinv[:, None]


_mm_call = pl.pallas_call(
    _mm_body,
    grid=(GRID,),
    in_specs=[
        pl.BlockSpec((MM_BLK, FEAT), lambda i: (i, 0)),
        pl.BlockSpec((FEAT, FEAT), lambda i: (0, 0)),
        pl.BlockSpec((MM_BLK, NW), lambda i: (i, 0)),
    ],
    out_specs=pl.BlockSpec((MM_BLK, FEAT), lambda i: (i, 0)),
    out_shape=jax.ShapeDtypeStruct((N_NODES, FEAT), jnp.float32),
)


def _fin_body(p_ref, degp_ref, b_ref, o_ref):
    deg = jnp.sum(degp_ref[...], axis=1)
    dinv = lax.rsqrt(deg)
    ssum = p_ref[0] + p_ref[1]
    o_ref[...] = jnp.maximum(ssum * dinv[:, None] + b_ref[...], 0.0)


_fin_call = pl.pallas_call(
    _fin_body,
    grid=(GRID,),
    in_specs=[
        pl.BlockSpec((2, MM_BLK, FEAT), lambda i: (0, i, 0)),
        pl.BlockSpec((MM_BLK, NW), lambda i: (i, 0)),
        pl.BlockSpec((1, FEAT), lambda i: (0, 0)),
    ],
    out_specs=pl.BlockSpec((MM_BLK, FEAT), lambda i: (i, 0)),
    out_shape=jax.ShapeDtypeStruct((N_NODES, FEAT), jnp.float32),
)


def kernel(x, edge_index, W, b):
    n = x.shape[0]
    loops = jnp.arange(n, dtype=jnp.int32)
    rows_all = jnp.concatenate([edge_index[0].astype(jnp.int32), loops])
    cols_all = jnp.concatenate([edge_index[1].astype(jnp.int32), loops])
    pad = TOTAL_SLOTS - rows_all.shape[0]
    rows_p = jnp.concatenate([rows_all, jnp.zeros((pad,), jnp.int32)])
    cols_p = jnp.concatenate([cols_all, jnp.full((pad,), n, jnp.int32)])
    rows_p = rows_p.reshape(NW, CHUNKS, CHUNK)
    cols_p = cols_p.reshape(NW, CHUNKS, CHUNK)
    rc = jnp.stack([rows_p, cols_p], axis=2).reshape(NW, 2 * CHUNKS, CHUNK)

    deg_call, edge_call = _sc_calls()
    degp = deg_call(cols_p).T
    hp = _mm_call(x, W, degp)
    part = edge_call(rc, hp)
    return _fin_call(part, degp, b.reshape(1, FEAT))


# speedup vs baseline: 1.0444x; 1.0444x over previous
import functools

import jax
import jax.numpy as jnp
from jax import lax
from jax.experimental import pallas as pl
from jax.experimental.pallas import tpu as pltpu
from jax.experimental.pallas import tpu_sc as plsc

N_NODES = 10000
FEAT = 128
NW = 32
CHUNK = 128
CHUNKS = 84
BLK = 4
NBLOCKS = CHUNKS // BLK
PER_TILE = CHUNKS * CHUNK
TOTAL_SLOTS = NW * PER_TILE
NB = 10240
ROWS_PER_TILE = NB // 16
MM_BLK = 1000
GRID = N_NODES // MM_BLK

def _deg_body(cols_hbm, degp_hbm, col_v, hist_v):
    c = lax.axis_index("c")
    s = lax.axis_index("s")
    w = s * 2 + c
    pltpu.sync_copy(cols_hbm.at[w], col_v)

    def zero(i, carry):
        hist_v[pl.ds(i * 16, 16)] = jnp.zeros((16,), jnp.float32)
        return carry

    lax.fori_loop(0, NB // 16, zero, 0)

    ones = jnp.ones((16,), jnp.float32)

    def body(i, carry):
        j = i // (CHUNK // 16)
        l = i % (CHUNK // 16)
        idx = col_v[j, pl.ds(l * 16, 16)]
        plsc.addupdate_scatter(hist_v, [idx], ones)
        return carry

    lax.fori_loop(0, CHUNKS * (CHUNK // 16), body, 0)
    pltpu.sync_copy(hist_v, degp_hbm.at[w])


def _edge_body(rows_hbm, cols_hbm, h_hbm, part_hbm, row_v, col_v, buf, acc, gs):
    c = lax.axis_index("c")
    s = lax.axis_index("s")
    w = s * 2 + c
    pltpu.sync_copy(rows_hbm.at[w], row_v)
    pltpu.sync_copy(cols_hbm.at[w], col_v)

    def zero(i, carry):
        buf[i // 8, pl.ds((i % 8) * 16, 16)] = jnp.zeros((16,), jnp.float32)
        return carry

    lax.fori_loop(0, CHUNK * 8, zero, 0)
    for t in range(ROWS_PER_TILE // CHUNK):
        pltpu.sync_copy(buf, acc.at[pl.ds(s * ROWS_PER_TILE + t * CHUNK, CHUNK)])
    plsc.subcore_barrier()

    def body(j, carry):
        pltpu.async_copy(h_hbm.at[row_v.at[j]], buf, gs).wait()
        pltpu.sync_copy(buf, acc.at[col_v.at[j]], add=True)
        return carry

    lax.fori_loop(0, CHUNKS, body, 0)
    plsc.subcore_barrier()
    pltpu.sync_copy(
        acc.at[pl.ds(s * ROWS_PER_TILE, ROWS_PER_TILE)],
        part_hbm.at[c].at[pl.ds(s * ROWS_PER_TILE, ROWS_PER_TILE)],
    )


@functools.lru_cache(maxsize=None)
def _sc_calls():
    mesh = plsc.VectorSubcoreMesh(core_axis_name="c", subcore_axis_name="s")
    params = pltpu.CompilerParams(needs_layout_passes=False)
    deg_call = functools.partial(
        pl.kernel,
        out_type=jax.ShapeDtypeStruct((NW, NB), jnp.float32),
        mesh=mesh,
        compiler_params=params,
        scratch_types=[
            pltpu.VMEM((CHUNKS, CHUNK), jnp.int32),
            pltpu.VMEM((NB,), jnp.float32),
        ],
    )(_deg_body)
    edge_call = functools.partial(
        pl.kernel,
        out_type=jax.ShapeDtypeStruct((2, NB, FEAT), jnp.float32),
        mesh=mesh,
        compiler_params=params,
        scratch_types=[
            pltpu.VMEM((CHUNKS, CHUNK), jnp.int32),
            pltpu.VMEM((CHUNKS, CHUNK), jnp.int32),
            pltpu.VMEM((CHUNK, FEAT), jnp.float32),
            pltpu.VMEM_SHARED((NB, FEAT), jnp.float32),
            pltpu.SemaphoreType.DMA,
        ],
    )(_edge_body)
    return deg_call, edge_call


def _mm_body(x_ref, w_ref, degp_ref, h_ref):
    deg = jnp.sum(degp_ref[...], axis=1)
    dinv = lax.rsqrt(deg)
    h = jnp.dot(x_ref[...], w_ref[...], preferred_element_type=jnp.float32)
    h_ref[...] = h * dinv[:, None]


_mm_call = pl.pallas_call(
    _mm_body,
    grid=(GRID,),
    in_specs=[
        pl.BlockSpec((MM_BLK, FEAT), lambda i: (i, 0)),
        pl.BlockSpec((FEAT, FEAT), lambda i: (0, 0)),
        pl.BlockSpec((MM_BLK, NW), lambda i: (i, 0)),
    ],
    out_specs=pl.BlockSpec((MM_BLK, FEAT), lambda i: (i, 0)),
    out_shape=jax.ShapeDtypeStruct((N_NODES, FEAT), jnp.float32),
)


def _fin_body(p_ref, degp_ref, b_ref, o_ref):
    deg = jnp.sum(degp_ref[...], axis=1)
    dinv = lax.rsqrt(deg)
    ssum = p_ref[0] + p_ref[1]
    o_ref[...] = jnp.maximum(ssum * dinv[:, None] + b_ref[...], 0.0)


_fin_call = pl.pallas_call(
    _fin_body,
    grid=(GRID,),
    in_specs=[
        pl.BlockSpec((2, MM_BLK, FEAT), lambda i: (0, i, 0)),
        pl.BlockSpec((MM_BLK, NW), lambda i: (i, 0)),
        pl.BlockSpec((1, FEAT), lambda i: (0, 0)),
    ],
    out_specs=pl.BlockSpec((MM_BLK, FEAT), lambda i: (i, 0)),
    out_shape=jax.ShapeDtypeStruct((N_NODES, FEAT), jnp.float32),
)


def kernel(x, edge_index, W, b):
    n = x.shape[0]
    loops = jnp.arange(n, dtype=jnp.int32)
    rows_all = jnp.concatenate([edge_index[0].astype(jnp.int32), loops])
    cols_all = jnp.concatenate([edge_index[1].astype(jnp.int32), loops])
    pad = TOTAL_SLOTS - rows_all.shape[0]
    rows_p = jnp.concatenate([rows_all, jnp.zeros((pad,), jnp.int32)])
    cols_p = jnp.concatenate([cols_all, jnp.full((pad,), n, jnp.int32)])
    rows_p = rows_p.reshape(NW, CHUNKS, CHUNK)
    cols_p = cols_p.reshape(NW, CHUNKS, CHUNK)
    deg_call, edge_call = _sc_calls()
    degp = deg_call(cols_p).T
    hp = _mm_call(x, W, degp)
    part = edge_call(rows_p, cols_p, hp)
    return _fin_call(part, degp, b.reshape(1, FEAT))


# speedup vs baseline: 2.5636x; 2.4548x over previous
import functools

import jax
import jax.numpy as jnp
from jax import lax
from jax.experimental import pallas as pl
from jax.experimental.pallas import tpu as pltpu
from jax.experimental.pallas import tpu_sc as plsc

N_NODES = 10000
FEAT = 128
NW = 32
CHUNK = 128
CHUNKS = 81
PER_TILE = CHUNKS * CHUNK
TOTAL_SLOTS = NW * PER_TILE
NB = 10240
ROWS_PER_TILE = NB // 16
MM_BLK = 1000
GRID = N_NODES // MM_BLK

def _deg_body(cols_hbm, degp_hbm, col_v, hist_v):
    c = lax.axis_index("c")
    s = lax.axis_index("s")
    w = s * 2 + c
    pltpu.sync_copy(cols_hbm.at[w], col_v)

    def zero(i, carry):
        hist_v[pl.ds(i * 16, 16)] = jnp.zeros((16,), jnp.float32)
        return carry

    lax.fori_loop(0, NB // 16, zero, 0)

    ones = jnp.ones((16,), jnp.float32)

    def body(i, carry):
        j = i // (CHUNK // 16)
        l = i % (CHUNK // 16)
        idx = col_v[j, pl.ds(l * 16, 16)]
        plsc.addupdate_scatter(hist_v, [idx], ones)
        return carry

    lax.fori_loop(0, CHUNKS * (CHUNK // 16), body, 0)
    pltpu.sync_copy(hist_v, degp_hbm.at[w])


def _edge_body(rows_hbm, cols_hbm, h_hbm, part_hbm, row_v, col_v, buf, acc, gs):
    c = lax.axis_index("c")
    s = lax.axis_index("s")
    w = s * 2 + c
    pltpu.sync_copy(rows_hbm.at[w], row_v)
    pltpu.sync_copy(cols_hbm.at[w], col_v)

    def zero(i, carry):
        buf[i // 8, pl.ds((i % 8) * 16, 16)] = jnp.zeros((16,), jnp.float32)
        return carry

    lax.fori_loop(0, CHUNK * 8, zero, 0)
    for t in range(ROWS_PER_TILE // CHUNK):
        pltpu.sync_copy(buf, acc.at[pl.ds(s * ROWS_PER_TILE + t * CHUNK, CHUNK)])
    plsc.subcore_barrier()

    def body(j, carry):
        pltpu.async_copy(h_hbm.at[row_v.at[j]], buf, gs).wait()
        pltpu.sync_copy(buf, acc.at[col_v.at[j]], add=True)
        return carry

    lax.fori_loop(0, CHUNKS, body, 0)
    plsc.subcore_barrier()
    pltpu.sync_copy(
        acc.at[pl.ds(s * ROWS_PER_TILE, ROWS_PER_TILE)],
        part_hbm.at[c].at[pl.ds(s * ROWS_PER_TILE, ROWS_PER_TILE)],
    )


@functools.lru_cache(maxsize=None)
def _sc_calls():
    mesh = plsc.VectorSubcoreMesh(core_axis_name="c", subcore_axis_name="s")
    params = pltpu.CompilerParams(needs_layout_passes=False)
    deg_call = functools.partial(
        pl.kernel,
        out_type=jax.ShapeDtypeStruct((NW, NB), jnp.float32),
        mesh=mesh,
        compiler_params=params,
        scratch_types=[
            pltpu.VMEM((CHUNKS, CHUNK), jnp.int32),
            pltpu.VMEM((NB,), jnp.float32),
        ],
    )(_deg_body)
    edge_call = functools.partial(
        pl.kernel,
        out_type=jax.ShapeDtypeStruct((2, NB, FEAT), jnp.float32),
        mesh=mesh,
        compiler_params=params,
        scratch_types=[
            pltpu.VMEM((CHUNKS, CHUNK), jnp.int32),
            pltpu.VMEM((CHUNKS, CHUNK), jnp.int32),
            pltpu.VMEM((CHUNK, FEAT), jnp.float32),
            pltpu.VMEM_SHARED((NB, FEAT), jnp.float32),
            pltpu.SemaphoreType.DMA,
        ],
    )(_edge_body)
    return deg_call, edge_call


def _mm_body(x_ref, w_ref, degp_ref, h_ref):
    deg = jnp.sum(degp_ref[...], axis=1)
    dinv = lax.rsqrt(deg)
    h = jnp.dot(x_ref[...], w_ref[...], preferred_element_type=jnp.float32)
    h_ref[...] = h * dinv[:, None]


_mm_call = pl.pallas_call(
    _mm_body,
    grid=(GRID,),
    in_specs=[
        pl.BlockSpec((MM_BLK, FEAT), lambda i: (i, 0)),
        pl.BlockSpec((FEAT, FEAT), lambda i: (0, 0)),
        pl.BlockSpec((MM_BLK, NW), lambda i: (i, 0)),
    ],
    out_specs=pl.BlockSpec((MM_BLK, FEAT), lambda i: (i, 0)),
    out_shape=jax.ShapeDtypeStruct((N_NODES, FEAT), jnp.float32),
)


def _fin_body(p_ref, degp_ref, b_ref, o_ref):
    deg = jnp.sum(degp_ref[...], axis=1)
    dinv = lax.rsqrt(deg)
    ssum = p_ref[0] + p_ref[1]
    o_ref[...] = jnp.maximum(ssum * dinv[:, None] + b_ref[...], 0.0)


_fin_call = pl.pallas_call(
    _fin_body,
    grid=(GRID,),
    in_specs=[
        pl.BlockSpec((2, MM_BLK, FEAT), lambda i: (0, i, 0)),
        pl.BlockSpec((MM_BLK, NW), lambda i: (i, 0)),
        pl.BlockSpec((1, FEAT), lambda i: (0, 0)),
    ],
    out_specs=pl.BlockSpec((MM_BLK, FEAT), lambda i: (i, 0)),
    out_shape=jax.ShapeDtypeStruct((N_NODES, FEAT), jnp.float32),
)


def kernel(x, edge_index, W, b):
    n = x.shape[0]
    loops = jnp.arange(n, dtype=jnp.int32)
    rows_all = jnp.concatenate([edge_index[0].astype(jnp.int32), loops])
    cols_all = jnp.concatenate([edge_index[1].astype(jnp.int32), loops])
    pad = TOTAL_SLOTS - rows_all.shape[0]
    pad_cols = n + (jnp.arange(pad, dtype=jnp.int32) % (NB - n))
    rows_p = jnp.concatenate([rows_all, jnp.zeros((pad,), jnp.int32)])
    cols_p = jnp.concatenate([cols_all, pad_cols])
    rows_p = rows_p.reshape(NW, CHUNKS, CHUNK)
    cols_p = cols_p.reshape(NW, CHUNKS, CHUNK)
    deg_call, edge_call = _sc_calls()
    degp = deg_call(cols_p).T
    hp = _mm_call(x, W, degp)
    part = edge_call(rows_p, cols_p, hp)
    return _fin_call(part, degp, b.reshape(1, FEAT))
